# baseline (speedup 1.0000x reference)
import functools

import jax
import jax.numpy as jnp
from jax import lax
from jax.experimental import pallas as pl
from jax.experimental.pallas import tpu as pltpu
from jax.experimental.pallas import tpu_sc as plsc


def _gather_sc(table, indices):
    B = indices.shape[0]
    V, D = table.shape
    info = plsc.get_sparse_core_info()
    nw = info.num_cores * info.num_subcores
    b_per_w = B // nw

    mesh = plsc.VectorSubcoreMesh(core_axis_name="c", subcore_axis_name="s")

    @functools.partial(
        pl.kernel,
        mesh=mesh,
        out_type=jax.ShapeDtypeStruct((B, D), jnp.float32),
        scratch_types=[
            pltpu.VMEM((b_per_w,), jnp.int32),
            pltpu.VMEM((b_per_w, D), jnp.float32),
            pltpu.SemaphoreType.DMA,
        ],
    )
    def gather_kernel(table_hbm, idx_hbm, out_hbm, idx_v, obuf, sem):
        wid = lax.axis_index("s") * info.num_cores + lax.axis_index("c")
        base = wid * b_per_w
        pltpu.sync_copy(idx_hbm.at[pl.ds(base, b_per_w)], idx_v)

        def body(j, _):
            v = idx_v[pl.ds(j * 16, 16)]
            for l in range(16):
                pltpu.async_copy(
                    table_hbm.at[v[l]], obuf.at[j * 16 + l], sem)
            return 0

        lax.fori_loop(0, b_per_w // 16, body, 0)
        pltpu.make_async_copy(
            out_hbm.at[pl.ds(base, b_per_w)], obuf, sem).wait()
        pltpu.sync_copy(obuf, out_hbm.at[pl.ds(base, b_per_w)])

    return gather_kernel(table, indices)


def _mlp_tc(x, W1, b1, W2, b2, blk):
    B, D = x.shape
    H1 = W1.shape[1]
    H2 = W2.shape[1]

    def body(x_ref, w1_ref, b1_ref, w2_ref, b2_ref, o_ref):
        h = jnp.dot(x_ref[...], w1_ref[...], preferred_element_type=jnp.float32)
        h = jnp.maximum(h + b1_ref[...], 0.0)
        o = jnp.dot(h, w2_ref[...], preferred_element_type=jnp.float32)
        o_ref[...] = jnp.maximum(o + b2_ref[...], 0.0)

    return pl.pallas_call(
        body,
        grid=(B // blk,),
        in_specs=[
            pl.BlockSpec((blk, D), lambda i: (i, 0)),
            pl.BlockSpec((D, H1), lambda i: (0, 0)),
            pl.BlockSpec((1, H1), lambda i: (0, 0)),
            pl.BlockSpec((H1, H2), lambda i: (0, 0)),
            pl.BlockSpec((1, H2), lambda i: (0, 0)),
        ],
        out_specs=pl.BlockSpec((blk, H2), lambda i: (i, 0)),
        out_shape=jax.ShapeDtypeStruct((B, H2), jnp.float32),
    )(x, W1, b1, W2, b2)


def kernel(indices, table, W1, b1, W2, b2):
    x = _gather_sc(table, indices.astype(jnp.int32))
    return _mlp_tc(
        x,
        W1,
        b1.reshape(1, -1),
        W2,
        b2.reshape(1, -1),
        blk=2048,
    )


# speedup vs baseline: 1.2388x; 1.2388x over previous
import functools

import jax
import jax.numpy as jnp
from jax import lax
from jax.experimental import pallas as pl
from jax.experimental.pallas import tpu as pltpu
from jax.experimental.pallas import tpu_sc as plsc


def _transpose_tc(tableT, blk):
    D, V = tableT.shape
    grid = (V + blk - 1) // blk

    def body(x_ref, o_ref):
        o_ref[...] = x_ref[...].T

    return pl.pallas_call(
        body,
        grid=(grid,),
        in_specs=[pl.BlockSpec((D, blk), lambda i: (0, i))],
        out_specs=pl.BlockSpec((blk, D), lambda i: (i, 0)),
        out_shape=jax.ShapeDtypeStruct((V, D), jnp.float32),
    )(tableT)


def _gather_sc(table, indices):
    V, D = table.shape
    B = indices.shape[0]
    info = plsc.get_sparse_core_info()
    nw = info.num_cores * info.num_subcores
    b_per_w = B // nw

    mesh = plsc.VectorSubcoreMesh(core_axis_name="c", subcore_axis_name="s")

    @functools.partial(
        pl.kernel,
        mesh=mesh,
        out_type=jax.ShapeDtypeStruct((B, D), jnp.float32),
        scratch_types=[
            pltpu.VMEM((b_per_w,), jnp.int32),
            pltpu.VMEM((b_per_w, D), jnp.float32),
            pltpu.SemaphoreType.DMA,
        ],
    )
    def gather_kernel(table_hbm, idx_hbm, out_hbm, idx_v, obuf, sem):
        wid = lax.axis_index("s") * info.num_cores + lax.axis_index("c")
        base = wid * b_per_w
        pltpu.sync_copy(idx_hbm.at[pl.ds(base, b_per_w)], idx_v)

        def body(j, _):
            v = idx_v[pl.ds(j * 16, 16)]
            for l in range(16):
                pltpu.async_copy(
                    table_hbm.at[v[l]], obuf.at[j * 16 + l], sem)
            return 0

        lax.fori_loop(0, b_per_w // 16, body, 0)
        pltpu.make_async_copy(
            out_hbm.at[pl.ds(base, b_per_w)], obuf, sem).wait()
        pltpu.sync_copy(obuf, out_hbm.at[pl.ds(base, b_per_w)])

    return gather_kernel(table, indices)


def _mlp_tc(x, W1, b1, W2, b2, blk):
    B, D = x.shape
    H1 = W1.shape[1]
    H2 = W2.shape[1]

    def body(x_ref, w1_ref, b1_ref, w2_ref, b2_ref, o_ref):
        h = jnp.dot(x_ref[...], w1_ref[...],
                    preferred_element_type=jnp.float32)
        h = jnp.maximum(h + b1_ref[...], 0.0)
        o = jnp.dot(h, w2_ref[...],
                    preferred_element_type=jnp.float32)
        o_ref[...] = jnp.maximum(o + b2_ref[...], 0.0)

    return pl.pallas_call(
        body,
        grid=(B // blk,),
        in_specs=[
            pl.BlockSpec((blk, D), lambda i: (i, 0)),
            pl.BlockSpec((D, H1), lambda i: (0, 0)),
            pl.BlockSpec((1, H1), lambda i: (0, 0)),
            pl.BlockSpec((H1, H2), lambda i: (0, 0)),
            pl.BlockSpec((1, H2), lambda i: (0, 0)),
        ],
        out_specs=pl.BlockSpec((blk, H2), lambda i: (i, 0)),
        out_shape=jax.ShapeDtypeStruct((B, H2), jnp.float32),
    )(x, W1, b1, W2, b2)


def kernel(indices, table, W1, b1, W2, b2):
    tableT = table.T
    table_rm = _transpose_tc(tableT, blk=8192)
    x = _gather_sc(table_rm, indices.astype(jnp.int32))
    return _mlp_tc(
        x,
        W1,
        b1.reshape(1, -1),
        W2,
        b2.reshape(1, -1),
        blk=2048,
    )


# speedup vs baseline: 1.2424x; 1.0029x over previous
import functools

import jax
import jax.numpy as jnp
from jax import lax
from jax.experimental import pallas as pl
from jax.experimental.pallas import tpu as pltpu
from jax.experimental.pallas import tpu_sc as plsc

_BLK = 8192


def _pack_tc(tableT):
    D, V = tableT.shape
    grid = (V + _BLK - 1) // _BLK
    half = _BLK // 2

    def body(x_ref, o_ref):
        y = x_ref[...].T
        o_ref[...] = jnp.concatenate([y[:half], y[half:]], axis=1)

    return pl.pallas_call(
        body,
        grid=(grid,),
        in_specs=[pl.BlockSpec((D, _BLK), lambda i: (0, i))],
        out_specs=pl.BlockSpec((half, 2 * D), lambda i: (i, 0)),
        out_shape=jax.ShapeDtypeStruct((grid * half, 2 * D), jnp.float32),
    )(tableT)


def _gather_sc(table2, indices):
    G, D2 = table2.shape
    B = indices.shape[0]
    info = plsc.get_sparse_core_info()
    nw = info.num_cores * info.num_subcores
    b_per_w = B // nw
    half = _BLK // 2

    mesh = plsc.VectorSubcoreMesh(core_axis_name="c", subcore_axis_name="s")

    @functools.partial(
        pl.kernel,
        mesh=mesh,
        out_type=jax.ShapeDtypeStruct((B, D2), jnp.float32),
        scratch_types=[
            pltpu.VMEM((b_per_w,), jnp.int32),
            pltpu.VMEM((b_per_w, D2), jnp.float32),
            pltpu.SemaphoreType.DMA,
        ],
    )
    def gather_kernel(table_hbm, idx_hbm, out_hbm, idx_v, obuf, sem):
        wid = lax.axis_index("s") * info.num_cores + lax.axis_index("c")
        base = wid * b_per_w
        pltpu.sync_copy(idx_hbm.at[pl.ds(base, b_per_w)], idx_v)

        def body(j, _):
            v = idx_v[pl.ds(j * 16, 16)]
            pos = jnp.bitwise_or(
                lax.shift_left(lax.shift_right_logical(v, 13), 12),
                jnp.bitwise_and(v, half - 1))
            for l in range(16):
                pltpu.async_copy(
                    table_hbm.at[pos[l]], obuf.at[j * 16 + l], sem)
            return 0

        lax.fori_loop(0, b_per_w // 16, body, 0)
        pltpu.make_async_copy(
            out_hbm.at[pl.ds(base, b_per_w)], obuf, sem).wait()
        pltpu.sync_copy(obuf, out_hbm.at[pl.ds(base, b_per_w)])

    return gather_kernel(table2, indices)


def _mlp_tc(x2, sel, W1, b1, W2, b2, blk):
    B, D2 = x2.shape
    D = D2 // 2
    H1 = W1.shape[1]
    H2 = W2.shape[1]

    def body(x_ref, s_ref, w1_ref, b1_ref, w2_ref, b2_ref, o_ref):
        s = s_ref[...]
        x = x_ref[:, :D] * (1.0 - s) + x_ref[:, D:] * s
        h = jnp.dot(x, w1_ref[...], preferred_element_type=jnp.float32)
        h = jnp.maximum(h + b1_ref[...], 0.0)
        o = jnp.dot(h, w2_ref[...], preferred_element_type=jnp.float32)
        o_ref[...] = jnp.maximum(o + b2_ref[...], 0.0)

    return pl.pallas_call(
        body,
        grid=(B // blk,),
        in_specs=[
            pl.BlockSpec((blk, D2), lambda i: (i, 0)),
            pl.BlockSpec((blk, 1), lambda i: (i, 0)),
            pl.BlockSpec((D, H1), lambda i: (0, 0)),
            pl.BlockSpec((1, H1), lambda i: (0, 0)),
            pl.BlockSpec((H1, H2), lambda i: (0, 0)),
            pl.BlockSpec((1, H2), lambda i: (0, 0)),
        ],
        out_specs=pl.BlockSpec((blk, H2), lambda i: (i, 0)),
        out_shape=jax.ShapeDtypeStruct((B, H2), jnp.float32),
    )(x2, sel, W1, b1, W2, b2)


def kernel(indices, table, W1, b1, W2, b2):
    idx = indices.astype(jnp.int32)
    tableT = table.T
    table2 = _pack_tc(tableT)
    x2 = _gather_sc(table2, idx)
    sel = ((idx >> 12) & 1).astype(jnp.float32).reshape(-1, 1)
    return _mlp_tc(
        x2,
        sel,
        W1,
        b1.reshape(1, -1),
        W2,
        b2.reshape(1, -1),
        blk=2048,
    )


# speedup vs baseline: 1.4739x; 1.1863x over previous
import functools

import jax
import jax.numpy as jnp
from jax import lax
from jax.experimental import pallas as pl
from jax.experimental.pallas import tpu as pltpu
from jax.experimental.pallas import tpu_sc as plsc

_BLK = 32768
_SHIFT = _BLK.bit_length() - 1


def _pack_tc(tableT):
    D, V = tableT.shape
    grid = (V + _BLK - 1) // _BLK
    half = _BLK // 2

    def body(x_ref, o_ref):
        y = x_ref[...].T
        o_ref[...] = jnp.concatenate([y[:half], y[half:]], axis=1)

    return pl.pallas_call(
        body,
        grid=(grid,),
        in_specs=[pl.BlockSpec((D, _BLK), lambda i: (0, i))],
        out_specs=pl.BlockSpec((half, 2 * D), lambda i: (i, 0)),
        out_shape=jax.ShapeDtypeStruct((grid * half, 2 * D), jnp.float32),
    )(tableT)


def _gather_sc(table2, indices):
    G, D2 = table2.shape
    B = indices.shape[0]
    info = plsc.get_sparse_core_info()
    nw = info.num_cores * info.num_subcores
    b_per_w = B // nw
    half = _BLK // 2

    mesh = plsc.VectorSubcoreMesh(core_axis_name="c", subcore_axis_name="s")

    @functools.partial(
        pl.kernel,
        mesh=mesh,
        out_type=jax.ShapeDtypeStruct((B, D2), jnp.float32),
        scratch_types=[
            pltpu.VMEM((b_per_w,), jnp.int32),
            pltpu.VMEM((b_per_w, D2), jnp.float32),
            pltpu.SemaphoreType.DMA,
        ],
    )
    def gather_kernel(table_hbm, idx_hbm, out_hbm, idx_v, obuf, sem):
        wid = lax.axis_index("s") * info.num_cores + lax.axis_index("c")
        base = wid * b_per_w
        pltpu.sync_copy(idx_hbm.at[pl.ds(base, b_per_w)], idx_v)

        def body(j, _):
            v = idx_v[pl.ds(j * 16, 16)]
            pos = jnp.bitwise_or(
                lax.shift_left(lax.shift_right_logical(v, _SHIFT), _SHIFT - 1),
                jnp.bitwise_and(v, half - 1))
            for l in range(16):
                pltpu.async_copy(
                    table_hbm.at[pos[l]], obuf.at[j * 16 + l], sem)
            return 0

        lax.fori_loop(0, b_per_w // 16, body, 0)
        pltpu.make_async_copy(
            out_hbm.at[pl.ds(base, b_per_w)], obuf, sem).wait()
        pltpu.sync_copy(obuf, out_hbm.at[pl.ds(base, b_per_w)])

    return gather_kernel(table2, indices)


def _mlp_tc(x2, sel, W1, b1, W2, b2, blk):
    B, D2 = x2.shape
    D = D2 // 2
    H1 = W1.shape[1]
    H2 = W2.shape[1]

    def body(x_ref, s_ref, w1_ref, b1_ref, w2_ref, b2_ref, o_ref):
        s = s_ref[...]
        x = x_ref[:, :D] * (1.0 - s) + x_ref[:, D:] * s
        h = jnp.dot(x, w1_ref[...], preferred_element_type=jnp.float32)
        h = jnp.maximum(h + b1_ref[...], 0.0)
        o = jnp.dot(h, w2_ref[...], preferred_element_type=jnp.float32)
        o_ref[...] = jnp.maximum(o + b2_ref[...], 0.0)

    return pl.pallas_call(
        body,
        grid=(B // blk,),
        in_specs=[
            pl.BlockSpec((blk, D2), lambda i: (i, 0)),
            pl.BlockSpec((blk, 1), lambda i: (i, 0)),
            pl.BlockSpec((D, H1), lambda i: (0, 0)),
            pl.BlockSpec((1, H1), lambda i: (0, 0)),
            pl.BlockSpec((H1, H2), lambda i: (0, 0)),
            pl.BlockSpec((1, H2), lambda i: (0, 0)),
        ],
        out_specs=pl.BlockSpec((blk, H2), lambda i: (i, 0)),
        out_shape=jax.ShapeDtypeStruct((B, H2), jnp.float32),
    )(x2, sel, W1, b1, W2, b2)


def kernel(indices, table, W1, b1, W2, b2):
    idx = indices.astype(jnp.int32)
    tableT = table.T
    table2 = _pack_tc(tableT)
    x2 = _gather_sc(table2, idx)
    sel = ((idx >> (_SHIFT - 1)) & 1).astype(jnp.float32).reshape(-1, 1)
    return _mlp_tc(
        x2,
        sel,
        W1,
        b1.reshape(1, -1),
        W2,
        b2.reshape(1, -1),
        blk=2048,
    )


# speedup vs baseline: 1.6523x; 1.1211x over previous
import functools

import jax
import jax.numpy as jnp
from jax import lax
from jax.experimental import pallas as pl
from jax.experimental.pallas import tpu as pltpu
from jax.experimental.pallas import tpu_sc as plsc

_BLK = 32768
_Q = _BLK // 4
_QSHIFT = _Q.bit_length() - 1
_BSHIFT = _BLK.bit_length() - 1


def _pack_tc(tableT):
    D, V = tableT.shape
    grid = (V + _BLK - 1) // _BLK

    def body(x_ref, o_ref):
        y = x_ref[...].T
        bits = lax.bitcast_convert_type(y, jnp.int32)
        hi_mask = jnp.int32(-65536)
        q = [bits[k * _Q:(k + 1) * _Q] for k in range(4)]
        w01 = jnp.bitwise_or(jnp.bitwise_and(q[0], hi_mask),
                             lax.shift_right_logical(q[1], 16))
        w23 = jnp.bitwise_or(jnp.bitwise_and(q[2], hi_mask),
                             lax.shift_right_logical(q[3], 16))
        packed = jnp.concatenate([w01, w23], axis=1)
        o_ref[...] = lax.bitcast_convert_type(packed, jnp.float32)

    return pl.pallas_call(
        body,
        grid=(grid,),
        in_specs=[pl.BlockSpec((D, _BLK), lambda i: (0, i))],
        out_specs=pl.BlockSpec((_Q, 2 * D), lambda i: (i, 0)),
        out_shape=jax.ShapeDtypeStruct((grid * _Q, 2 * D), jnp.float32),
    )(tableT)


def _gather_sc(table2, indices):
    G, D2 = table2.shape
    B = indices.shape[0]
    info = plsc.get_sparse_core_info()
    nw = info.num_cores * info.num_subcores
    b_per_w = B // nw

    mesh = plsc.VectorSubcoreMesh(core_axis_name="c", subcore_axis_name="s")

    @functools.partial(
        pl.kernel,
        mesh=mesh,
        out_type=jax.ShapeDtypeStruct((B, D2), jnp.float32),
        scratch_types=[
            pltpu.VMEM((b_per_w,), jnp.int32),
            pltpu.VMEM((b_per_w, D2), jnp.float32),
            pltpu.SemaphoreType.DMA,
        ],
    )
    def gather_kernel(table_hbm, idx_hbm, out_hbm, idx_v, obuf, sem):
        wid = lax.axis_index("s") * info.num_cores + lax.axis_index("c")
        base = wid * b_per_w
        pltpu.sync_copy(idx_hbm.at[pl.ds(base, b_per_w)], idx_v)

        def body(j, _):
            v = idx_v[pl.ds(j * 16, 16)]
            pos = jnp.bitwise_or(
                lax.shift_left(lax.shift_right_logical(v, _BSHIFT), _QSHIFT),
                jnp.bitwise_and(v, _Q - 1))
            for l in range(16):
                pltpu.async_copy(
                    table_hbm.at[pos[l]], obuf.at[j * 16 + l], sem)
            return 0

        lax.fori_loop(0, b_per_w // 16, body, 0)
        pltpu.make_async_copy(
            out_hbm.at[pl.ds(base, b_per_w)], obuf, sem).wait()
        pltpu.sync_copy(obuf, out_hbm.at[pl.ds(base, b_per_w)])

    return gather_kernel(table2, indices)


def _mlp_tc(x2, sel_col, sel_lo, W1, b1, W2, b2, blk):
    B, D2 = x2.shape
    D = D2 // 2
    H1 = W1.shape[1]
    H2 = W2.shape[1]

    def body(x_ref, sc_ref, sl_ref, w1_ref, b1_ref, w2_ref, b2_ref, o_ref):
        u = lax.bitcast_convert_type(x_ref[...], jnp.int32)
        col = sc_ref[...] != 0
        low = sl_ref[...] != 0
        ua = jnp.where(col, u[:, D:], u[:, :D])
        ub = jnp.where(low, lax.shift_left(ua, 16),
                       jnp.bitwise_and(ua, jnp.int32(-65536)))
        x = lax.bitcast_convert_type(ub, jnp.float32)
        h = jnp.dot(x, w1_ref[...], preferred_element_type=jnp.float32)
        h = jnp.maximum(h + b1_ref[...], 0.0)
        o = jnp.dot(h, w2_ref[...], preferred_element_type=jnp.float32)
        o_ref[...] = jnp.maximum(o + b2_ref[...], 0.0)

    return pl.pallas_call(
        body,
        grid=(B // blk,),
        in_specs=[
            pl.BlockSpec((blk, D2), lambda i: (i, 0)),
            pl.BlockSpec((blk, 1), lambda i: (i, 0)),
            pl.BlockSpec((blk, 1), lambda i: (i, 0)),
            pl.BlockSpec((D, H1), lambda i: (0, 0)),
            pl.BlockSpec((1, H1), lambda i: (0, 0)),
            pl.BlockSpec((H1, H2), lambda i: (0, 0)),
            pl.BlockSpec((1, H2), lambda i: (0, 0)),
        ],
        out_specs=pl.BlockSpec((blk, H2), lambda i: (i, 0)),
        out_shape=jax.ShapeDtypeStruct((B, H2), jnp.float32),
    )(x2, sel_col, sel_lo, W1, b1, W2, b2)


def kernel(indices, table, W1, b1, W2, b2):
    idx = indices.astype(jnp.int32)
    tableT = table.T
    table2 = _pack_tc(tableT)
    x2 = _gather_sc(table2, idx)
    quarter = (idx >> _QSHIFT) & 3
    sel_col = (quarter >= 2).astype(jnp.int32).reshape(-1, 1)
    sel_lo = (quarter & 1).astype(jnp.int32).reshape(-1, 1)
    return _mlp_tc(
        x2,
        sel_col,
        sel_lo,
        W1,
        b1.reshape(1, -1),
        W2,
        b2.reshape(1, -1),
        blk=2048,
    )
